# baseline (speedup 1.0000x reference)
import functools

import numpy as np
import jax
import jax.numpy as jnp
from jax import lax
from jax.experimental import pallas as pl
from jax.experimental.pallas import tpu as pltpu
from jax.experimental.pallas import tpu_sc as plsc

VOCAB = 1000000
EMBED_DIM = 64
BATCH = 4
SEQ_LEN = 2048

NW = 32
TOTAL = BATCH * SEQ_LEN
PER_W = TOTAL // NW
W_PER_SEQ = SEQ_LEN // PER_W
LANES = 128
NBUF = 8
HALF = PER_W // 2


def _sinusoidal_pe_np(seq_len, d_model):
    position = np.arange(seq_len, dtype=np.float32)[:, None]
    div_term = np.exp(
        np.arange(0, d_model, 2, dtype=np.float32) * (-np.log(10000.0) / d_model))
    pe = np.zeros((seq_len, d_model), dtype=np.float32)
    pe[:, 0::2] = np.sin(position * div_term)
    pe[:, 1::2] = np.cos(position * div_term)
    return pe


_PE_NP = _sinusoidal_pe_np(SEQ_LEN, EMBED_DIM).reshape(W_PER_SEQ, PER_W, EMBED_DIM)


@functools.partial(
    pl.kernel,
    out_type=jax.ShapeDtypeStruct((NW, PER_W, EMBED_DIM), jnp.float32),
    mesh=plsc.VectorSubcoreMesh(core_axis_name="c", subcore_axis_name="s"),
    compiler_params=pltpu.CompilerParams(
        use_tc_tiling_on_sc=True, needs_layout_passes=False),
    scratch_types=[
        pltpu.VMEM((PER_W + 16,), jnp.int32),
        pltpu.VMEM((NBUF, EMBED_DIM, LANES), jnp.float32),
        pltpu.VMEM((HALF, EMBED_DIM), jnp.float32),
        pltpu.VMEM((HALF, EMBED_DIM), jnp.float32),
        [pltpu.SemaphoreType.DMA] * NBUF,
        [pltpu.SemaphoreType.DMA] * NBUF,
        pltpu.SemaphoreType.DMA,
    ],
)
def _emb_sc(x_hbm, pe_hbm, tabt_hbm, out_hbm,
            idx_v, slab_v, pe_v, rows_v, gsems, hsems, psem):
    wid = lax.axis_index("s") * 2 + lax.axis_index("c")
    wslot = lax.rem(wid, W_PER_SEQ)
    pltpu.sync_copy(x_hbm.at[wid], idx_v.at[pl.ds(0, PER_W)])
    cpp = pltpu.async_copy(pe_hbm.at[wslot, pl.ds(0, HALF)], pe_v, psem)

    def fire(v, buf):
        c = lax.shift_right_logical(v, 7)
        off = pl.multiple_of(c * LANES, LANES)
        pltpu.async_copy(
            tabt_hbm.at[pl.ds(0, 32), pl.ds(off, LANES)],
            slab_v.at[buf, pl.ds(0, 32)], gsems[buf])
        pltpu.async_copy(
            tabt_hbm.at[pl.ds(32, 32), pl.ds(off, LANES)],
            slab_v.at[buf, pl.ds(32, 32)], hsems[buf])

    vec0 = idx_v[pl.ds(0, 16)]
    for j in range(NBUF):
        fire(vec0[j], j)
    cpp.wait()

    def make_body(half):
        def body(grp, carry):
            vec_cur = idx_v[pl.ds(grp * 16, 16)]
            vec_next = idx_v[pl.ds(grp * 16 + 16, 16)]
            for j in range(16):
                buf = j % NBUF
                t = grp * 16 + j
                tl = t - half * HALF
                pltpu.make_async_copy(
                    tabt_hbm.at[pl.ds(0, 32), pl.ds(0, LANES)],
                    slab_v.at[buf, pl.ds(0, 32)], gsems[buf]).wait()
                pltpu.make_async_copy(
                    tabt_hbm.at[pl.ds(32, 32), pl.ds(0, LANES)],
                    slab_v.at[buf, pl.ds(32, 32)], hsems[buf]).wait()
                l_vec = jnp.full((16,), vec_cur[j] & (LANES - 1), dtype=jnp.int32)
                for k in range(EMBED_DIM // 16):
                    d_vec = lax.iota(jnp.int32, 16) + (16 * k)
                    g = plsc.load_gather(slab_v.at[buf], [d_vec, l_vec])
                    sl = pl.ds(16 * k, 16)
                    rows_v[tl, sl] = g + pe_v[tl, sl]

                v_ahead = vec_cur[j + NBUF] if j + NBUF < 16 else vec_next[j + NBUF - 16]

                @pl.when(t + NBUF < PER_W)
                def _():
                    fire(v_ahead, buf)

            return carry
        return body

    g_half = HALF // 16
    for half in range(2):
        lax.fori_loop(half * g_half, (half + 1) * g_half, make_body(half), 0)
        pltpu.sync_copy(rows_v, out_hbm.at[wid, pl.ds(half * HALF, HALF)])
        if half == 0:
            pltpu.async_copy(
                pe_hbm.at[wslot, pl.ds(HALF, HALF)], pe_v, psem).wait()


def kernel(x, token_embedding):
    x_w = x.reshape(NW, PER_W).astype(jnp.int32)
    tab_t = token_embedding.T
    out = _emb_sc(x_w, jnp.asarray(_PE_NP), tab_t)
    return out.reshape(BATCH, SEQ_LEN, EMBED_DIM)


# speedup vs baseline: 1.1101x; 1.1101x over previous
import functools

import numpy as np
import jax
import jax.numpy as jnp
from jax import lax
from jax.experimental import pallas as pl
from jax.experimental.pallas import tpu as pltpu
from jax.experimental.pallas import tpu_sc as plsc

VOCAB = 1000000
EMBED_DIM = 64
BATCH = 4
SEQ_LEN = 2048

NW = 32
TOTAL = BATCH * SEQ_LEN
PER_W = TOTAL // NW
W_PER_SEQ = SEQ_LEN // PER_W
LANES = 128
NBUF = 8
HALF = PER_W // 2


def _sinusoidal_pe_np(seq_len, d_model):
    position = np.arange(seq_len, dtype=np.float32)[:, None]
    div_term = np.exp(
        np.arange(0, d_model, 2, dtype=np.float32) * (-np.log(10000.0) / d_model))
    pe = np.zeros((seq_len, d_model), dtype=np.float32)
    pe[:, 0::2] = np.sin(position * div_term)
    pe[:, 1::2] = np.cos(position * div_term)
    return pe


_PE_NP = _sinusoidal_pe_np(SEQ_LEN, EMBED_DIM).reshape(W_PER_SEQ, PER_W, EMBED_DIM)


@functools.partial(
    pl.kernel,
    out_type=jax.ShapeDtypeStruct((NW, PER_W, EMBED_DIM), jnp.float32),
    mesh=plsc.VectorSubcoreMesh(core_axis_name="c", subcore_axis_name="s"),
    compiler_params=pltpu.CompilerParams(
        use_tc_tiling_on_sc=True, needs_layout_passes=False),
    scratch_types=[
        pltpu.VMEM((PER_W + 16,), jnp.int32),
        pltpu.VMEM((NBUF, EMBED_DIM, LANES), jnp.float32),
        pltpu.VMEM((HALF, EMBED_DIM), jnp.float32),
        pltpu.VMEM((HALF, EMBED_DIM), jnp.float32),
        [pltpu.SemaphoreType.DMA] * NBUF,
        pltpu.SemaphoreType.DMA,
    ],
)
def _emb_sc(x_hbm, pe_hbm, tabt_hbm, out_hbm,
            idx_v, slab_v, pe_v, rows_v, gsems, psem):
    wid = lax.axis_index("s") * 2 + lax.axis_index("c")
    wslot = lax.rem(wid, W_PER_SEQ)
    pltpu.sync_copy(x_hbm.at[wid], idx_v.at[pl.ds(0, PER_W)])
    cpp = pltpu.async_copy(pe_hbm.at[wslot, pl.ds(0, HALF)], pe_v, psem)

    def fire(v, buf):
        c = lax.shift_right_logical(v, 7)
        off = pl.multiple_of(c * LANES, LANES)
        pltpu.async_copy(
            tabt_hbm.at[:, pl.ds(off, LANES)], slab_v.at[buf], gsems[buf])

    vec0 = idx_v[pl.ds(0, 16)]
    for j in range(NBUF):
        fire(vec0[j], j)
    cpp.wait()

    def make_body(half):
        def body(grp, carry):
            vec_cur = idx_v[pl.ds(grp * 16, 16)]
            vec_next = idx_v[pl.ds(grp * 16 + 16, 16)]
            for j in range(16):
                buf = j % NBUF
                t = grp * 16 + j
                tl = t - half * HALF
                pltpu.make_async_copy(
                    tabt_hbm.at[:, pl.ds(0, LANES)], slab_v.at[buf], gsems[buf]).wait()
                l_vec = jnp.full((16,), vec_cur[j] & (LANES - 1), dtype=jnp.int32)
                for k in range(EMBED_DIM // 16):
                    d_vec = lax.iota(jnp.int32, 16) + (16 * k)
                    g = plsc.load_gather(slab_v.at[buf], [d_vec, l_vec])
                    sl = pl.ds(16 * k, 16)
                    rows_v[tl, sl] = g + pe_v[tl, sl]

                v_ahead = vec_cur[j + NBUF] if j + NBUF < 16 else vec_next[j + NBUF - 16]

                @pl.when(t + NBUF < PER_W)
                def _():
                    fire(v_ahead, buf)

            return carry
        return body

    g_half = HALF // 16
    for half in range(2):
        lax.fori_loop(half * g_half, (half + 1) * g_half, make_body(half), 0)
        pltpu.sync_copy(rows_v, out_hbm.at[wid, pl.ds(half * HALF, HALF)])
        if half == 0:
            pltpu.async_copy(
                pe_hbm.at[wslot, pl.ds(HALF, HALF)], pe_v, psem).wait()


def kernel(x, token_embedding):
    x_w = x.reshape(NW, PER_W).astype(jnp.int32)
    tab_t = token_embedding.T
    out = _emb_sc(x_w, jnp.asarray(_PE_NP), tab_t)
    return out.reshape(BATCH, SEQ_LEN, EMBED_DIM)
